# Initial kernel scaffold; baseline (speedup 1.0000x reference)
import jax
import jax.numpy as jnp
from jax.experimental import pallas as pl


def kernel(x, edge_index, Wl1, Wr1, b1, Wl2, Wr2, b2, Wl3, Wr3, b3):
    raise NotImplementedError("write your pallas kernel here")


# speedup vs baseline: 4.4842x; 4.4842x over previous
import functools

import jax
import jax.numpy as jnp
from jax import lax
from jax.experimental import pallas as pl
from jax.experimental.pallas import tpu as pltpu
from jax.experimental.pallas import tpu_sc as plsc

N = 10000
D = 128
E = 320000

NP = 10240
NW = 32
CHUNK = 128
KCH = 80
EPW = CHUNK * KCH
EP = NW * EPW
DUMMY_DST = N + 8
RPS = NP // 16

_f32 = jnp.float32
_HI = lax.Precision.HIGHEST


def _seg_partial_sums(table, src3, dst3, W):
    nv = W // 16
    mesh = plsc.VectorSubcoreMesh(core_axis_name="c", subcore_axis_name="s")

    @functools.partial(
        pl.kernel,
        out_type=jax.ShapeDtypeStruct((2, NP, W), _f32),
        mesh=mesh,
        scratch_types=[
            pltpu.VMEM((KCH, CHUNK), jnp.int32),
            pltpu.VMEM((KCH, CHUNK), jnp.int32),
            pltpu.VMEM((CHUNK, W), _f32),
            pltpu.VMEM_SHARED((NP, W), _f32),
            pltpu.SemaphoreType.DMA,
        ],
        compiler_params=pltpu.CompilerParams(use_tc_tiling_on_sc=False),
    )
    def k(table_hbm, src_hbm, dst_hbm, out_hbm, src_v, dst_v, rows_v, acc, sem):
        c = lax.axis_index("c")
        s = lax.axis_index("s")
        wid = s * 2 + c
        pltpu.sync_copy(src_hbm.at[wid], src_v)
        pltpu.sync_copy(dst_hbm.at[wid], dst_v)

        zz = jnp.zeros((16,), _f32)

        def zrow(i, carry):
            for j in range(nv):
                rows_v[i, pl.ds(j * 16, 16)] = zz
            return carry

        lax.fori_loop(0, CHUNK, zrow, 0)
        for t in range(RPS // CHUNK):
            pltpu.sync_copy(rows_v, acc.at[pl.ds(s * RPS + t * CHUNK, CHUNK)])
        plsc.subcore_barrier()

        def chunk_body(g, carry):
            pltpu.async_copy(table_hbm.at[src_v.at[g]], rows_v, sem).wait()
            pltpu.sync_copy(rows_v, acc.at[dst_v.at[g]], add=True)
            return carry

        lax.fori_loop(0, KCH, chunk_body, 0)
        plsc.subcore_barrier()

        for t in range(RPS // CHUNK):
            off = s * RPS + t * CHUNK
            pltpu.sync_copy(acc.at[pl.ds(off, CHUNK)],
                            out_hbm.at[c, pl.ds(off, CHUNK)])

    return k(table, src3, dst3)


BLK = 2048
GRID = NP // BLK


def _rows(shape):
    nd = len(shape)
    if nd == 1:
        return pl.BlockSpec((BLK,), lambda i: (i,))
    blk = shape[:-2] + (BLK, shape[-1])
    idx = {2: (lambda i: (i, 0)), 3: (lambda i: (0, i, 0))}[nd]
    return pl.BlockSpec(blk, idx)


def _whole(shape):
    return pl.BlockSpec(shape, lambda i: (0,) * len(shape))


def _d1(x_ref, wl_ref, wr_ref, b_ref, paug_ref, r_ref):
    x = x_ref[...]
    p = jnp.dot(x, wl_ref[...], preferred_element_type=_f32, precision=_HI)
    paug_ref[...] = jnp.concatenate([p, jnp.ones((BLK, 16), _f32)], axis=1)
    r_ref[...] = (
        jnp.dot(x, wr_ref[...], preferred_element_type=_f32, precision=_HI)
        + b_ref[...]
    )


def _d2(parts_ref, r1_ref, wl_ref, wr_ref, b_ref, p2_ref, r2_ref, rdeg_ref):
    ps = parts_ref[0] + parts_ref[1]
    deg = jnp.maximum(ps[:, 128:129], 1.0)
    rdeg = 1.0 / deg
    h1 = jnp.maximum(ps[:, :128] * rdeg + r1_ref[...], 0.0)
    p2_ref[...] = jnp.dot(h1, wl_ref[...], preferred_element_type=_f32,
                          precision=_HI)
    r2_ref[...] = (
        jnp.dot(h1, wr_ref[...], preferred_element_type=_f32, precision=_HI)
        + b_ref[...]
    )
    rdeg_ref[...] = rdeg


def _d3(parts_ref, r2_ref, rdeg_ref, wl_ref, wr_ref, b_ref, p3_ref, t3_ref):
    ps = parts_ref[0] + parts_ref[1]
    h2 = jnp.maximum(ps * rdeg_ref[...] + r2_ref[...], 0.0)
    s3 = jnp.dot(h2, wl_ref[...], preferred_element_type=_f32, precision=_HI)
    p3_ref[...] = jnp.broadcast_to(s3, (BLK, 16))
    t3_ref[...] = (
        jnp.dot(h2, wr_ref[...], preferred_element_type=_f32, precision=_HI)
        + b_ref[...]
    )


def _d4(parts_ref, t3_ref, rdeg_ref, out_ref):
    m = (parts_ref[0, :, 0:1] + parts_ref[1, :, 0:1]) * rdeg_ref[...] + t3_ref[...]
    out_ref[...] = m[:, 0]


def _sds(shape):
    return jax.ShapeDtypeStruct(shape, _f32)


def kernel(x, edge_index, Wl1, Wr1, b1, Wl2, Wr2, b2, Wl3, Wr3, b3):
    x_pad = jnp.pad(x, ((0, NP - N), (0, 0)))
    src = edge_index[0]
    dst = edge_index[1]
    pad_e = EP - E
    src3 = jnp.concatenate(
        [src, jnp.zeros((pad_e,), jnp.int32)]).reshape(NW, KCH, CHUNK)
    dst3 = jnp.concatenate(
        [dst, jnp.full((pad_e,), DUMMY_DST, jnp.int32)]).reshape(NW, KCH, CHUNK)

    paug, r1 = pl.pallas_call(
        _d1,
        grid=(GRID,),
        in_specs=[_rows((NP, 128)), _whole((D, 128)), _whole((D, 128)),
                  _whole((128,))],
        out_specs=(_rows((NP, 144)), _rows((NP, 128))),
        out_shape=(_sds((NP, 144)), _sds((NP, 128))),
    )(x_pad, Wl1, Wr1, b1)

    parts1 = _seg_partial_sums(paug, src3, dst3, 144)

    p2, r2, rdeg = pl.pallas_call(
        _d2,
        grid=(GRID,),
        in_specs=[_rows((2, NP, 144)), _rows((NP, 128)), _whole((128, 128)),
                  _whole((128, 128)), _whole((128,))],
        out_specs=(_rows((NP, 128)), _rows((NP, 128)), _rows((NP, 1))),
        out_shape=(_sds((NP, 128)), _sds((NP, 128)), _sds((NP, 1))),
    )(parts1, r1, Wl2, Wr2, b2)

    parts2 = _seg_partial_sums(p2, src3, dst3, 128)

    p3, t3 = pl.pallas_call(
        _d3,
        grid=(GRID,),
        in_specs=[_rows((2, NP, 128)), _rows((NP, 128)), _rows((NP, 1)),
                  _whole((128, 1)), _whole((128, 1)), _whole((1,))],
        out_specs=(_rows((NP, 16)), _rows((NP, 1))),
        out_shape=(_sds((NP, 16)), _sds((NP, 1))),
    )(parts2, r2, rdeg, Wl3, Wr3, b3)

    parts3 = _seg_partial_sums(p3, src3, dst3, 16)

    out = pl.pallas_call(
        _d4,
        grid=(GRID,),
        in_specs=[_rows((2, NP, 16)), _rows((NP, 1)), _rows((NP, 1))],
        out_specs=_rows((NP,)),
        out_shape=_sds((NP,)),
    )(parts3, t3, rdeg)

    return out[:N]
